# baseline (speedup 1.0000x reference)
import jax
import jax.numpy as jnp
from jax import lax
from jax.experimental import pallas as pl
from jax.experimental.pallas import tpu as pltpu
from jax.experimental.pallas import tpu_sc as plsc

_B = 4096
_D = 64
_DP = 128
_M = 8
_INV_T = 2.0
_NC, _NS = 1, 16
_NW = _NC * _NS
_R = _B // _NW
_PPREF = 512
_OW = 16


def _threefry2x32(k0, k1, x0, x1):
    import numpy as np

    def rotl(x, r):
        return ((x << np.uint32(r)) | (x >> np.uint32(32 - r))).astype(np.uint32)

    ks0, ks1 = np.uint32(k0), np.uint32(k1)
    ks2 = np.uint32(ks0 ^ ks1 ^ np.uint32(0x1BD11BDA))
    rot1 = (13, 15, 26, 6)
    rot2 = (17, 29, 16, 24)
    x0 = (x0 + ks0).astype(np.uint32)
    x1 = (x1 + ks1).astype(np.uint32)

    def rounds(x0, x1, rots):
        for r in rots:
            x0 = (x0 + x1).astype(np.uint32)
            x1 = rotl(x1, r)
            x1 = (x1 ^ x0).astype(np.uint32)
        return x0, x1

    for i, (rots, ka, kb) in enumerate([
            (rot1, ks1, ks2), (rot2, ks2, ks0), (rot1, ks0, ks1),
            (rot2, ks1, ks2), (rot1, ks2, ks0)]):
        x0, x1 = rounds(x0, x1, rots)
        x0 = (x0 + ka).astype(np.uint32)
        x1 = (x1 + kb + np.uint32(i + 1)).astype(np.uint32)
    return x0, x1


def _np_gumbel(kd, n):
    import numpy as np

    i = np.arange(n, dtype=np.uint32)
    y0, y1 = _threefry2x32(kd[0], kd[1], np.zeros(n, np.uint32), i)
    bits = (y0 ^ y1).astype(np.uint32)
    fb = ((bits >> np.uint32(9)) | np.uint32(0x3F800000)).astype(np.uint32)
    f = fb.view(np.float32) - np.float32(1.0)
    tiny = np.float32(np.finfo(np.float32).tiny)
    u = np.maximum(tiny, f * (np.float32(1.0) - tiny) + tiny)
    return -np.log(-np.log(u))


def _perm_consts():
    import numpy as np

    kp = (1832780943, 270669613)
    kn = (64467757, 2916123636)
    gp = _np_gumbel(kp, _B * _B).reshape(_B, _B)
    gn = _np_gumbel(kn, _B * _B).reshape(_B, _B)
    pp = np.argsort(-gp, axis=1, kind="stable").astype(np.int32)
    pn = np.argsort(-gn, axis=1, kind="stable").astype(np.int32)
    pp16 = np.ascontiguousarray(pp).astype(np.uint16).view(np.int32)
    pnc = np.ascontiguousarray(pn[:, :16]).reshape(-1)
    return pp16, pnc, pn


_PP16, _PNC, _PN = _perm_consts()


def _rsqrt(x):
    i = plsc.bitcast(x, jnp.int32)
    y = plsc.bitcast(jnp.int32(0x5F3759DF) - (i >> 1), jnp.float32)
    for _ in range(3):
        y = y * (1.5 - 0.5 * x * y * y)
    return y


def _sc_body(feats, labels, pp16, pnc, pn, out,
             lab_v, ppre_v, pnpre_v, ptmp_v, cidx_v, valid_v,
             gath_v, outb_v, sem_a, sem_b, sem_c):
    i32 = jnp.int32
    iota = lax.iota(i32, 16)
    wid = lax.axis_index("s")
    base = pl.multiple_of(wid * _R, _R)

    cps = [
        pltpu.async_copy(labels, lab_v, sem_c),
        pltpu.async_copy(pp16.at[pl.ds(base, _R), pl.ds(0, _PPREF // 2)],
                         ppre_v, sem_c),
        pltpu.async_copy(pnc.at[pl.ds(base * 16, _R * 16)], pnpre_v, sem_c),
    ]
    for c in cps:
        c.wait()

    for g in range(_R // 16):
        plsc.store_scatter(cidx_v, [g * 160 + iota], base + g * 16 + iota)

    _NBIG = jnp.int32(1 << 20)
    _SEG = 128

    def unpack_lo(v):
        return v & 0xFFFF

    def unpack_hi(v):
        return (v >> 16) & 0xFFFF

    def pair_fn(i, carry):
        rows_meta = []
        for s in range(2):
            r = 2 * i + s
            anchor = base + r
            meta = dict(
                r=r,
                anchor=anchor,
                avec=jnp.full((16,), anchor, i32),
                rvec=jnp.full((16,), r, i32),
                gvec=jnp.full((16,), r // 16, i32),
                lvec=jnp.full((16,), r % 16, i32),
            )
            meta["mylab"] = plsc.load_gather(lab_v, [meta["avec"]])
            rows_meta.append(meta)

        def match_keys(md, vwords, colbase, t):
            lo = unpack_lo(vwords)
            hi = unpack_hi(vwords)
            llo = plsc.load_gather(lab_v, [lo])
            lhi = plsc.load_gather(lab_v, [hi])
            mlo = (llo == md["mylab"]) & (lo != md["avec"])
            mhi = (lhi == md["mylab"]) & (hi != md["avec"])
            klo = jnp.where(mlo, colbase + 32 * t + 2 * iota, _NBIG)
            khi = jnp.where(mhi, colbase + 32 * t + 2 * iota + 1, _NBIG)
            return jnp.minimum(klo, khi)

        def seg_cond(c):
            seg, m0, m1 = c
            return (seg < _PPREF // _SEG) & ((m0 == _NBIG) | (m1 == _NBIG))

        def seg_body(c):
            seg, m0, m1 = c
            mins = [m0, m1]
            for s in range(2):
                md = rows_meta[s]
                runmin = jnp.full((16,), _NBIG, i32)
                for t in range(_SEG // 32):
                    vw = plsc.load_gather(
                        ppre_v, [md["rvec"], seg * (_SEG // 2) + t * 16 + iota])
                    runmin = jnp.minimum(runmin,
                                         match_keys(md, vw, seg * _SEG, t))
                mins[s] = jnp.minimum(mins[s], jnp.min(runmin))
            return (seg + 1, mins[0], mins[1])

        _, min0, min1 = lax.while_loop(seg_cond, seg_body, (0, _NBIG, _NBIG))

        for s, poscol in ((0, min0), (1, min1)):
            md = rows_meta[s]
            pfound = (poscol < _NBIG).astype(i32)
            e = jnp.where(pfound == 1, poscol >> 1, 0)
            vw = plsc.load_gather(ppre_v, [md["rvec"],
                                           jnp.full((16,), e, i32)])
            val = jnp.where(poscol % 2 == 1, unpack_hi(vw), unpack_lo(vw))
            md["pfound"] = pfound
            md["pval"] = jnp.where(pfound == 1, jnp.max(val), 0)

        for md in rows_meta:
            anchor = md["anchor"]
            mylab = md["mylab"]
            avec = md["avec"]
            rvec = md["rvec"]
            a8 = pl.multiple_of((anchor // 8) * 8, 8)
            arvec = jnp.full((16,), anchor % 8, i32)

            def pfb_cond(c):
                col, found, _ = c
                return (found == 0) & (col < _B)

            def pfb_body(c, md=md, a8=a8, arvec=arvec):
                col, found, val = c
                pltpu.sync_copy(
                    pp16.at[pl.ds(a8, 8),
                            pl.ds(pl.multiple_of(col // 2, 128), 128)],
                    ptmp_v)
                runmin = jnp.full((16,), _NBIG, i32)
                for t in range(8):
                    vw = plsc.load_gather(ptmp_v, [arvec, t * 16 + iota])
                    runmin = jnp.minimum(runmin,
                                         match_keys(md, vw, col, t))
                fmin = jnp.min(runmin)
                f2 = (fmin < _NBIG).astype(i32)
                el = jnp.where(f2 == 1, (fmin - col) >> 1, 0)
                vw = plsc.load_gather(ptmp_v, [arvec,
                                               jnp.full((16,), el, i32)])
                v2 = jnp.where(fmin % 2 == 1, unpack_hi(vw), unpack_lo(vw))
                v2s = jnp.max(v2)
                return (col + 256, found | f2, jnp.where(f2 == 1, v2s, val))

            _, md["pfound"], md["pval"] = lax.while_loop(
                pfb_cond, pfb_body, (_PPREF, md["pfound"], md["pval"]))

            def nbody_once(cnt, vidx, want_diff=True, mylab=mylab,
                           gvec=md["gvec"], lvec=md["lvec"]):
                vlab = plsc.load_gather(lab_v, [vidx])
                m = (vlab != mylab) if want_diff else (vlab == mylab)
                rank = plsc.cumsum(m.astype(i32))
                sel = m & ((cnt + rank) <= _M)
                slot = jnp.where(sel, cnt + rank + 1, 2)
                plsc.store_scatter(cidx_v, [(gvec * 10 + slot) * 16 + lvec],
                                   vidx, mask=sel)
                return jnp.minimum(cnt + jnp.max(rank), _M)

            ncnt = nbody_once(jnp.int32(0),
                              plsc.load_gather(pnpre_v, [rvec * 16 + iota]))

            def nfb_cond(c):
                col, cnt = c
                return (cnt < _M) & (col < _B)

            def nfb_body(c, a8=a8, arvec=arvec, nbody_once=nbody_once):
                col, cnt = c
                pltpu.sync_copy(
                    pn.at[pl.ds(a8, 8),
                          pl.ds(pl.multiple_of(col, 128), 128)], ptmp_v)

                def inner(c2):
                    t, cnt2 = c2
                    vidx = plsc.load_gather(ptmp_v, [arvec, t * 16 + iota])
                    return (t + 1, nbody_once(cnt2, vidx))

                def inner_cond(c2):
                    t, cnt2 = c2
                    return (cnt2 < _M) & (t < 8)

                _, cnt = lax.while_loop(inner_cond, inner, (0, cnt))
                return (col + 128, cnt)

            ncnt0 = jnp.where(ncnt < _M, 0, ncnt)
            _, ncnt = lax.while_loop(nfb_cond, nfb_body, (0, ncnt0))
            anyneg = (ncnt > 0).astype(jnp.float32)

            def pad_cond(c):
                t, cnt = c
                return (cnt < _M) & (t < _B // 16)

            def pad_body(c, nbody_once=nbody_once):
                t, cnt = c
                return (t + 1, nbody_once(cnt, t * 16 + iota,
                                          want_diff=False))

            _, ncnt = lax.while_loop(pad_cond, pad_body, (0, ncnt))

            valid = md["pfound"].astype(jnp.float32) * anyneg
            lane0 = iota == 0
            plsc.store_scatter(cidx_v,
                               [(md["gvec"] * 10 + 1) * 16 + md["lvec"]],
                               jnp.full((16,), md["pval"], i32), mask=lane0)
            plsc.store_scatter(valid_v, [md["rvec"]],
                               jnp.full((16,), valid, jnp.float32),
                               mask=lane0)
        return carry

    lax.fori_loop(0, _R // 2, pair_fn, 0)

    zero16 = jnp.zeros((16,), jnp.float32)
    ngroups = _R // 16
    sems = (sem_a, sem_b)

    def fire(g):
        return [pltpu.async_copy(
            feats.at[cidx_v.at[pl.ds((g * 10 + m) * 16, 16)]],
            gath_v.at[g % 2, m], sems[g % 2]) for m in range(_M + 2)]

    pending = {0: fire(0), 1: fire(1)}
    for g in range(ngroups):
        buf = g % 2
        for c in pending.pop(g):
            c.wait()

        rows = g * 16 + iota
        bufv = jnp.full((16,), buf, i32)
        mvecs = [jnp.full((16,), m, i32) for m in range(_M + 2)]

        def dbody(d, carry, _bufv=bufv, _mvecs=mvecs):
            a2 = carry[0]
            accs = carry[1:10]
            c2s = carry[10:19]
            dv = jnp.full((16,), d, i32)
            a = plsc.load_gather(gath_v, [_bufv, _mvecs[0], iota, dv])
            out_accs = []
            out_c2s = []
            for k in range(9):
                b = plsc.load_gather(gath_v, [_bufv, _mvecs[k + 1], iota, dv])
                out_accs.append(accs[k] + a * b)
                out_c2s.append(c2s[k] + b * b)
            return (a2 + a * a, *out_accs, *out_c2s)

        res = lax.fori_loop(0, _D, dbody, tuple(zero16 for _ in range(19)))
        a2 = res[0]
        accs = res[1:10]
        c2s = res[10:19]

        ra = _rsqrt(jnp.maximum(a2, 1e-24))
        simv = [accs[k] * ra * _rsqrt(jnp.maximum(c2s[k], 1e-24))
                for k in range(9)]

        t1 = jnp.full((16,), -3.0e38, jnp.float32)
        t2 = t1
        t3 = t1
        for k in range(1, 9):
            v = simv[k]
            n1 = jnp.maximum(t1, v)
            v2 = jnp.minimum(t1, v)
            n2 = jnp.maximum(t2, v2)
            v3 = jnp.minimum(t2, v2)
            n3 = jnp.maximum(t3, v3)
            t1, t2, t3 = n1, n2, n3

        validv = plsc.load_gather(valid_v, [rows])
        cols = [simv[0], t1, t2, t3, validv]
        for c in range(_OW):
            vec = cols[c] if c < 5 else zero16
            plsc.store_scatter(outb_v, [iota, jnp.full((16,), c, i32)], vec)
        row0 = pl.multiple_of(base + g * 16, 16)
        pltpu.sync_copy(outb_v, out.at[pl.ds(row0, 16), :])

        if g + 2 < ngroups:
            pending[g + 2] = fire(g + 2)


_mesh = plsc.VectorSubcoreMesh(core_axis_name="c", subcore_axis_name="s",
                               num_cores=_NC, num_subcores=_NS)
_sc_select = pl.kernel(
    _sc_body,
    out_type=jax.ShapeDtypeStruct((_B, _OW), jnp.float32),
    mesh=_mesh,
    compiler_params=pltpu.CompilerParams(needs_layout_passes=False),
    scratch_types=[
        pltpu.VMEM((_B,), jnp.int32),
        pltpu.VMEM((_R, _PPREF // 2), jnp.int32),
        pltpu.VMEM((_R * 16,), jnp.int32),
        pltpu.VMEM((8, 128), jnp.int32),
        pltpu.VMEM(((_R // 16) * (_M + 2) * 16,), jnp.int32),
        pltpu.VMEM((_R,), jnp.float32),
        pltpu.VMEM((2, _M + 2, 16, _DP), jnp.float32),
        pltpu.VMEM((16, _OW), jnp.float32),
        pltpu.SemaphoreType.DMA,
        pltpu.SemaphoreType.DMA,
        pltpu.SemaphoreType.DMA,
    ],
)


def _loss_body(x_ref, o_ref):
    x = x_ref[...]
    l0 = x[:, 0:1] * _INV_T
    l1 = x[:, 1:2] * _INV_T
    l2 = x[:, 2:3] * _INV_T
    l3 = x[:, 3:4] * _INV_T
    v = x[:, 4:5]
    m = jnp.maximum(jnp.maximum(l0, l1), jnp.maximum(l2, l3))
    lse = m + jnp.log(jnp.exp(l0 - m) + jnp.exp(l1 - m)
                      + jnp.exp(l2 - m) + jnp.exp(l3 - m))
    losses = lse - l0
    nv = jnp.maximum(jnp.sum(v), 1.0)
    o_ref[...] = (jnp.sum(losses * v) / nv).reshape(1, 1)


_loss = pl.pallas_call(
    _loss_body,
    out_shape=jax.ShapeDtypeStruct((1, 1), jnp.float32),
)


def kernel(features, labels):
    labels = labels.reshape(-1).astype(jnp.int32)
    fpad = jnp.pad(features, ((0, 0), (0, _DP - _D)))
    sc = _sc_select(fpad, labels, _PP16, _PNC, _PN)
    return _loss(sc).reshape(())


# speedup vs baseline: 1.4056x; 1.4056x over previous
import jax
import jax.numpy as jnp
from jax import lax
from jax.experimental import pallas as pl
from jax.experimental.pallas import tpu as pltpu
from jax.experimental.pallas import tpu_sc as plsc

_B = 4096
_D = 64
_DP = 128
_M = 8
_INV_T = 2.0
_NC, _NS = 2, 16
_NW = _NC * _NS
_R = _B // _NW
_PPREF = 512
_OW = 16


def _threefry2x32(k0, k1, x0, x1):
    import numpy as np

    def rotl(x, r):
        return ((x << np.uint32(r)) | (x >> np.uint32(32 - r))).astype(np.uint32)

    ks0, ks1 = np.uint32(k0), np.uint32(k1)
    ks2 = np.uint32(ks0 ^ ks1 ^ np.uint32(0x1BD11BDA))
    rot1 = (13, 15, 26, 6)
    rot2 = (17, 29, 16, 24)
    x0 = (x0 + ks0).astype(np.uint32)
    x1 = (x1 + ks1).astype(np.uint32)

    def rounds(x0, x1, rots):
        for r in rots:
            x0 = (x0 + x1).astype(np.uint32)
            x1 = rotl(x1, r)
            x1 = (x1 ^ x0).astype(np.uint32)
        return x0, x1

    for i, (rots, ka, kb) in enumerate([
            (rot1, ks1, ks2), (rot2, ks2, ks0), (rot1, ks0, ks1),
            (rot2, ks1, ks2), (rot1, ks2, ks0)]):
        x0, x1 = rounds(x0, x1, rots)
        x0 = (x0 + ka).astype(np.uint32)
        x1 = (x1 + kb + np.uint32(i + 1)).astype(np.uint32)
    return x0, x1


def _np_gumbel(kd, n):
    import numpy as np

    i = np.arange(n, dtype=np.uint32)
    y0, y1 = _threefry2x32(kd[0], kd[1], np.zeros(n, np.uint32), i)
    bits = (y0 ^ y1).astype(np.uint32)
    fb = ((bits >> np.uint32(9)) | np.uint32(0x3F800000)).astype(np.uint32)
    f = fb.view(np.float32) - np.float32(1.0)
    tiny = np.float32(np.finfo(np.float32).tiny)
    u = np.maximum(tiny, f * (np.float32(1.0) - tiny) + tiny)
    return -np.log(-np.log(u))


def _perm_consts():
    import numpy as np

    kp = (1832780943, 270669613)
    kn = (64467757, 2916123636)
    gp = _np_gumbel(kp, _B * _B).reshape(_B, _B)
    gn = _np_gumbel(kn, _B * _B).reshape(_B, _B)
    pp = np.argsort(-gp, axis=1, kind="stable").astype(np.int32)
    pn = np.argsort(-gn, axis=1, kind="stable").astype(np.int32)
    pp16 = np.ascontiguousarray(pp).astype(np.uint16).view(np.int32)
    pnc = np.ascontiguousarray(pn[:, :16]).reshape(-1)
    return pp16, pnc, pn


_PP16, _PNC, _PN = _perm_consts()


def _rsqrt(x):
    i = plsc.bitcast(x, jnp.int32)
    y = plsc.bitcast(jnp.int32(0x5F3759DF) - (i >> 1), jnp.float32)
    for _ in range(3):
        y = y * (1.5 - 0.5 * x * y * y)
    return y


def _sc_body(feats, labels, pp16, pnc, pn, out,
             lab_v, ppre_v, pnpre_v, ptmp_v, cidx_v, valid_v,
             gath_v, outb_v, sem_a, sem_b, sem_c):
    i32 = jnp.int32
    iota = lax.iota(i32, 16)
    wid = lax.axis_index("s") * _NC + lax.axis_index("c")
    base = pl.multiple_of(wid * _R, _R)

    cps = [
        pltpu.async_copy(labels, lab_v, sem_c),
        pltpu.async_copy(pp16.at[pl.ds(base, _R), pl.ds(0, _PPREF // 2)],
                         ppre_v, sem_c),
        pltpu.async_copy(pnc.at[pl.ds(base * 16, _R * 16)], pnpre_v, sem_c),
    ]
    for c in cps:
        c.wait()

    for g in range(_R // 16):
        plsc.store_scatter(cidx_v, [g * 160 + iota], base + g * 16 + iota)

    _NBIG = jnp.int32(1 << 20)
    _SEG = 128

    def unpack_lo(v):
        return v & 0xFFFF

    def unpack_hi(v):
        return (v >> 16) & 0xFFFF

    def pair_fn(i, carry):
        rows_meta = []
        for s in range(2):
            r = 2 * i + s
            anchor = base + r
            meta = dict(
                r=r,
                anchor=anchor,
                avec=jnp.full((16,), anchor, i32),
                rvec=jnp.full((16,), r, i32),
                gvec=jnp.full((16,), r // 16, i32),
                lvec=jnp.full((16,), r % 16, i32),
            )
            meta["mylab"] = plsc.load_gather(lab_v, [meta["avec"]])
            rows_meta.append(meta)

        def match_keys(md, vwords, colbase, t):
            lo = unpack_lo(vwords)
            hi = unpack_hi(vwords)
            llo = plsc.load_gather(lab_v, [lo])
            lhi = plsc.load_gather(lab_v, [hi])
            mlo = (llo == md["mylab"]) & (lo != md["avec"])
            mhi = (lhi == md["mylab"]) & (hi != md["avec"])
            klo = jnp.where(mlo, colbase + 32 * t + 2 * iota, _NBIG)
            khi = jnp.where(mhi, colbase + 32 * t + 2 * iota + 1, _NBIG)
            return jnp.minimum(klo, khi)

        def seg_cond(c):
            seg, m0, m1 = c
            return (seg < _PPREF // _SEG) & ((m0 == _NBIG) | (m1 == _NBIG))

        def seg_body(c):
            seg, m0, m1 = c
            mins = [m0, m1]
            for s in range(2):
                md = rows_meta[s]
                runmin = jnp.full((16,), _NBIG, i32)
                for t in range(_SEG // 32):
                    vw = plsc.load_gather(
                        ppre_v, [md["rvec"], seg * (_SEG // 2) + t * 16 + iota])
                    runmin = jnp.minimum(runmin,
                                         match_keys(md, vw, seg * _SEG, t))
                mins[s] = jnp.minimum(mins[s], jnp.min(runmin))
            return (seg + 1, mins[0], mins[1])

        _, min0, min1 = lax.while_loop(seg_cond, seg_body, (0, _NBIG, _NBIG))

        for s, poscol in ((0, min0), (1, min1)):
            md = rows_meta[s]
            pfound = (poscol < _NBIG).astype(i32)
            e = jnp.where(pfound == 1, poscol >> 1, 0)
            vw = plsc.load_gather(ppre_v, [md["rvec"],
                                           jnp.full((16,), e, i32)])
            val = jnp.where(poscol % 2 == 1, unpack_hi(vw), unpack_lo(vw))
            md["pfound"] = pfound
            md["pval"] = jnp.where(pfound == 1, jnp.max(val), 0)

        for md in rows_meta:
            anchor = md["anchor"]
            mylab = md["mylab"]
            avec = md["avec"]
            rvec = md["rvec"]
            a8 = pl.multiple_of((anchor // 8) * 8, 8)
            arvec = jnp.full((16,), anchor % 8, i32)

            def pfb_cond(c):
                col, found, _ = c
                return (found == 0) & (col < _B)

            def pfb_body(c, md=md, a8=a8, arvec=arvec):
                col, found, val = c
                pltpu.sync_copy(
                    pp16.at[pl.ds(a8, 8),
                            pl.ds(pl.multiple_of(col // 2, 128), 128)],
                    ptmp_v)
                runmin = jnp.full((16,), _NBIG, i32)
                for t in range(8):
                    vw = plsc.load_gather(ptmp_v, [arvec, t * 16 + iota])
                    runmin = jnp.minimum(runmin,
                                         match_keys(md, vw, col, t))
                fmin = jnp.min(runmin)
                f2 = (fmin < _NBIG).astype(i32)
                el = jnp.where(f2 == 1, (fmin - col) >> 1, 0)
                vw = plsc.load_gather(ptmp_v, [arvec,
                                               jnp.full((16,), el, i32)])
                v2 = jnp.where(fmin % 2 == 1, unpack_hi(vw), unpack_lo(vw))
                v2s = jnp.max(v2)
                return (col + 256, found | f2, jnp.where(f2 == 1, v2s, val))

            _, md["pfound"], md["pval"] = lax.while_loop(
                pfb_cond, pfb_body, (_PPREF, md["pfound"], md["pval"]))

            def nbody_once(cnt, vidx, want_diff=True, mylab=mylab,
                           gvec=md["gvec"], lvec=md["lvec"]):
                vlab = plsc.load_gather(lab_v, [vidx])
                m = (vlab != mylab) if want_diff else (vlab == mylab)
                rank = plsc.cumsum(m.astype(i32))
                sel = m & ((cnt + rank) <= _M)
                slot = jnp.where(sel, cnt + rank + 1, 2)
                plsc.store_scatter(cidx_v, [(gvec * 10 + slot) * 16 + lvec],
                                   vidx, mask=sel)
                return jnp.minimum(cnt + jnp.max(rank), _M)

            ncnt = nbody_once(jnp.int32(0),
                              plsc.load_gather(pnpre_v, [rvec * 16 + iota]))

            def nfb_cond(c):
                col, cnt = c
                return (cnt < _M) & (col < _B)

            def nfb_body(c, a8=a8, arvec=arvec, nbody_once=nbody_once):
                col, cnt = c
                pltpu.sync_copy(
                    pn.at[pl.ds(a8, 8),
                          pl.ds(pl.multiple_of(col, 128), 128)], ptmp_v)

                def inner(c2):
                    t, cnt2 = c2
                    vidx = plsc.load_gather(ptmp_v, [arvec, t * 16 + iota])
                    return (t + 1, nbody_once(cnt2, vidx))

                def inner_cond(c2):
                    t, cnt2 = c2
                    return (cnt2 < _M) & (t < 8)

                _, cnt = lax.while_loop(inner_cond, inner, (0, cnt))
                return (col + 128, cnt)

            ncnt0 = jnp.where(ncnt < _M, 0, ncnt)
            _, ncnt = lax.while_loop(nfb_cond, nfb_body, (0, ncnt0))
            anyneg = (ncnt > 0).astype(jnp.float32)

            def pad_cond(c):
                t, cnt = c
                return (cnt < _M) & (t < _B // 16)

            def pad_body(c, nbody_once=nbody_once):
                t, cnt = c
                return (t + 1, nbody_once(cnt, t * 16 + iota,
                                          want_diff=False))

            _, ncnt = lax.while_loop(pad_cond, pad_body, (0, ncnt))

            valid = md["pfound"].astype(jnp.float32) * anyneg
            lane0 = iota == 0
            plsc.store_scatter(cidx_v,
                               [(md["gvec"] * 10 + 1) * 16 + md["lvec"]],
                               jnp.full((16,), md["pval"], i32), mask=lane0)
            plsc.store_scatter(valid_v, [md["rvec"]],
                               jnp.full((16,), valid, jnp.float32),
                               mask=lane0)
        return carry

    lax.fori_loop(0, _R // 2, pair_fn, 0)

    zero16 = jnp.zeros((16,), jnp.float32)
    ngroups = _R // 16
    sems = (sem_a, sem_b)

    def fire(g):
        return [pltpu.async_copy(
            feats.at[cidx_v.at[pl.ds((g * 10 + m) * 16, 16)]],
            gath_v.at[g % 2, m], sems[g % 2]) for m in range(_M + 2)]

    pending = {0: fire(0), 1: fire(1)}
    for g in range(ngroups):
        buf = g % 2
        for c in pending.pop(g):
            c.wait()

        rows = g * 16 + iota
        bufv = jnp.full((16,), buf, i32)
        mvecs = [jnp.full((16,), m, i32) for m in range(_M + 2)]

        def dbody(d, carry, _bufv=bufv, _mvecs=mvecs):
            a2 = carry[0]
            accs = carry[1:10]
            c2s = carry[10:19]
            dv = jnp.full((16,), d, i32)
            a = plsc.load_gather(gath_v, [_bufv, _mvecs[0], iota, dv])
            out_accs = []
            out_c2s = []
            for k in range(9):
                b = plsc.load_gather(gath_v, [_bufv, _mvecs[k + 1], iota, dv])
                out_accs.append(accs[k] + a * b)
                out_c2s.append(c2s[k] + b * b)
            return (a2 + a * a, *out_accs, *out_c2s)

        res = lax.fori_loop(0, _D, dbody, tuple(zero16 for _ in range(19)))
        a2 = res[0]
        accs = res[1:10]
        c2s = res[10:19]

        ra = _rsqrt(jnp.maximum(a2, 1e-24))
        simv = [accs[k] * ra * _rsqrt(jnp.maximum(c2s[k], 1e-24))
                for k in range(9)]

        t1 = jnp.full((16,), -3.0e38, jnp.float32)
        t2 = t1
        t3 = t1
        for k in range(1, 9):
            v = simv[k]
            n1 = jnp.maximum(t1, v)
            v2 = jnp.minimum(t1, v)
            n2 = jnp.maximum(t2, v2)
            v3 = jnp.minimum(t2, v2)
            n3 = jnp.maximum(t3, v3)
            t1, t2, t3 = n1, n2, n3

        validv = plsc.load_gather(valid_v, [rows])
        cols = [simv[0], t1, t2, t3, validv]
        for c in range(_OW):
            vec = cols[c] if c < 5 else zero16
            plsc.store_scatter(outb_v, [iota, jnp.full((16,), c, i32)], vec)
        row0 = pl.multiple_of(base + g * 16, 16)
        pltpu.sync_copy(outb_v, out.at[pl.ds(row0, 16), :])

        if g + 2 < ngroups:
            pending[g + 2] = fire(g + 2)


_mesh = plsc.VectorSubcoreMesh(core_axis_name="c", subcore_axis_name="s",
                               num_cores=_NC, num_subcores=_NS)
_sc_select = pl.kernel(
    _sc_body,
    out_type=jax.ShapeDtypeStruct((_B, _OW), jnp.float32),
    mesh=_mesh,
    compiler_params=pltpu.CompilerParams(needs_layout_passes=False),
    scratch_types=[
        pltpu.VMEM((_B,), jnp.int32),
        pltpu.VMEM((_R, _PPREF // 2), jnp.int32),
        pltpu.VMEM((_R * 16,), jnp.int32),
        pltpu.VMEM((8, 128), jnp.int32),
        pltpu.VMEM(((_R // 16) * (_M + 2) * 16,), jnp.int32),
        pltpu.VMEM((_R,), jnp.float32),
        pltpu.VMEM((2, _M + 2, 16, _DP), jnp.float32),
        pltpu.VMEM((16, _OW), jnp.float32),
        pltpu.SemaphoreType.DMA,
        pltpu.SemaphoreType.DMA,
        pltpu.SemaphoreType.DMA,
    ],
)


def _loss_body(x_ref, o_ref):
    x = x_ref[...]
    l0 = x[:, 0:1] * _INV_T
    l1 = x[:, 1:2] * _INV_T
    l2 = x[:, 2:3] * _INV_T
    l3 = x[:, 3:4] * _INV_T
    v = x[:, 4:5]
    m = jnp.maximum(jnp.maximum(l0, l1), jnp.maximum(l2, l3))
    lse = m + jnp.log(jnp.exp(l0 - m) + jnp.exp(l1 - m)
                      + jnp.exp(l2 - m) + jnp.exp(l3 - m))
    losses = lse - l0
    nv = jnp.maximum(jnp.sum(v), 1.0)
    o_ref[...] = (jnp.sum(losses * v) / nv).reshape(1, 1)


_loss = pl.pallas_call(
    _loss_body,
    out_shape=jax.ShapeDtypeStruct((1, 1), jnp.float32),
)


def kernel(features, labels):
    labels = labels.reshape(-1).astype(jnp.int32)
    fpad = jnp.pad(features, ((0, 0), (0, _DP - _D)))
    sc = _sc_select(fpad, labels, _PP16, _PNC, _PN)
    return _loss(sc).reshape(())


# speedup vs baseline: 1.4068x; 1.0009x over previous
import jax
import jax.numpy as jnp
from jax import lax
from jax.experimental import pallas as pl
from jax.experimental.pallas import tpu as pltpu
from jax.experimental.pallas import tpu_sc as plsc

_B = 4096
_D = 64
_DP = 128
_M = 8
_INV_T = 2.0
_NC, _NS = 2, 16
_NW = _NC * _NS
_R = _B // _NW
_PPREF = 512
_OW = 16


def _threefry2x32(k0, k1, x0, x1):
    import numpy as np

    def rotl(x, r):
        return ((x << np.uint32(r)) | (x >> np.uint32(32 - r))).astype(np.uint32)

    ks0, ks1 = np.uint32(k0), np.uint32(k1)
    ks2 = np.uint32(ks0 ^ ks1 ^ np.uint32(0x1BD11BDA))
    rot1 = (13, 15, 26, 6)
    rot2 = (17, 29, 16, 24)
    x0 = (x0 + ks0).astype(np.uint32)
    x1 = (x1 + ks1).astype(np.uint32)

    def rounds(x0, x1, rots):
        for r in rots:
            x0 = (x0 + x1).astype(np.uint32)
            x1 = rotl(x1, r)
            x1 = (x1 ^ x0).astype(np.uint32)
        return x0, x1

    for i, (rots, ka, kb) in enumerate([
            (rot1, ks1, ks2), (rot2, ks2, ks0), (rot1, ks0, ks1),
            (rot2, ks1, ks2), (rot1, ks2, ks0)]):
        x0, x1 = rounds(x0, x1, rots)
        x0 = (x0 + ka).astype(np.uint32)
        x1 = (x1 + kb + np.uint32(i + 1)).astype(np.uint32)
    return x0, x1


def _np_gumbel(kd, n):
    import numpy as np

    i = np.arange(n, dtype=np.uint32)
    y0, y1 = _threefry2x32(kd[0], kd[1], np.zeros(n, np.uint32), i)
    bits = (y0 ^ y1).astype(np.uint32)
    fb = ((bits >> np.uint32(9)) | np.uint32(0x3F800000)).astype(np.uint32)
    f = fb.view(np.float32) - np.float32(1.0)
    tiny = np.float32(np.finfo(np.float32).tiny)
    u = np.maximum(tiny, f * (np.float32(1.0) - tiny) + tiny)
    return -np.log(-np.log(u))


def _perm_consts():
    import numpy as np

    kp = (1832780943, 270669613)
    kn = (64467757, 2916123636)
    gp = _np_gumbel(kp, _B * _B).reshape(_B, _B)
    gn = _np_gumbel(kn, _B * _B).reshape(_B, _B)
    pp = np.argsort(-gp, axis=1, kind="stable").astype(np.int32)
    pn = np.argsort(-gn, axis=1, kind="stable").astype(np.int32)
    pp16 = np.ascontiguousarray(pp).astype(np.uint16).view(np.int32)
    pnc = np.ascontiguousarray(pn[:, :16]).reshape(-1)
    return pp16, pnc, pn


_PP16, _PNC, _PN = _perm_consts()


def _rsqrt(x):
    i = plsc.bitcast(x, jnp.int32)
    y = plsc.bitcast(jnp.int32(0x5F3759DF) - (i >> 1), jnp.float32)
    for _ in range(3):
        y = y * (1.5 - 0.5 * x * y * y)
    return y


def _sc_body(feats, labels, pp16, pnc, pn, out,
             lab_v, ppre_v, pnpre_v, ptmp_v, cidx_v, valid_v,
             gath_v, outb_v, sem_a, sem_b, sem_c):
    i32 = jnp.int32
    iota = lax.iota(i32, 16)
    wid = lax.axis_index("s") * _NC + lax.axis_index("c")
    base = pl.multiple_of(wid * _R, _R)

    cps = [
        pltpu.async_copy(labels, lab_v, sem_c),
        pltpu.async_copy(pp16.at[pl.ds(base, _R), pl.ds(0, _PPREF // 2)],
                         ppre_v, sem_c),
        pltpu.async_copy(pnc.at[pl.ds(base * 16, _R * 16)], pnpre_v, sem_c),
    ]
    for c in cps:
        c.wait()

    for g in range(_R // 16):
        plsc.store_scatter(cidx_v, [g * 160 + iota], base + g * 16 + iota)

    _NBIG = jnp.int32(1 << 20)
    _SEG = 128

    def unpack_lo(v):
        return v & 0xFFFF

    def unpack_hi(v):
        return (v >> 16) & 0xFFFF

    def pair_fn(i, carry):
        rows_meta = []
        for s in range(2):
            r = 2 * i + s
            anchor = base + r
            meta = dict(
                r=r,
                anchor=anchor,
                avec=jnp.full((16,), anchor, i32),
                rvec=jnp.full((16,), r, i32),
                gvec=jnp.full((16,), r // 16, i32),
                lvec=jnp.full((16,), r % 16, i32),
            )
            meta["mylab"] = plsc.load_gather(lab_v, [meta["avec"]])
            rows_meta.append(meta)

        def match_keys(md, vwords, colbase, t):
            lo = unpack_lo(vwords)
            hi = unpack_hi(vwords)
            llo = plsc.load_gather(lab_v, [lo])
            lhi = plsc.load_gather(lab_v, [hi])
            mlo = (llo == md["mylab"]) & (lo != md["avec"])
            mhi = (lhi == md["mylab"]) & (hi != md["avec"])
            klo = jnp.where(mlo, colbase + 32 * t + 2 * iota, _NBIG)
            khi = jnp.where(mhi, colbase + 32 * t + 2 * iota + 1, _NBIG)
            return jnp.minimum(klo, khi)

        def seg_cond(c):
            seg, m0, m1 = c
            return (seg < _PPREF // _SEG) & ((m0 == _NBIG) | (m1 == _NBIG))

        def seg_body(c):
            seg, m0, m1 = c
            mins = [m0, m1]
            for s in range(2):
                md = rows_meta[s]
                runmin = jnp.full((16,), _NBIG, i32)
                for t in range(_SEG // 32):
                    vw = plsc.load_gather(
                        ppre_v, [md["rvec"], seg * (_SEG // 2) + t * 16 + iota])
                    runmin = jnp.minimum(runmin,
                                         match_keys(md, vw, seg * _SEG, t))
                mins[s] = jnp.minimum(mins[s], jnp.min(runmin))
            return (seg + 1, mins[0], mins[1])

        _, min0, min1 = lax.while_loop(seg_cond, seg_body, (0, _NBIG, _NBIG))

        for s, poscol in ((0, min0), (1, min1)):
            md = rows_meta[s]
            pfound = (poscol < _NBIG).astype(i32)
            e = jnp.where(pfound == 1, poscol >> 1, 0)
            vw = plsc.load_gather(ppre_v, [md["rvec"],
                                           jnp.full((16,), e, i32)])
            val = jnp.where(poscol % 2 == 1, unpack_hi(vw), unpack_lo(vw))
            md["pfound"] = pfound
            md["pval"] = jnp.where(pfound == 1, jnp.max(val), 0)

        for md in rows_meta:
            anchor = md["anchor"]
            mylab = md["mylab"]
            avec = md["avec"]
            rvec = md["rvec"]
            a8 = pl.multiple_of((anchor // 8) * 8, 8)
            arvec = jnp.full((16,), anchor % 8, i32)

            def pfb_cond(c):
                col, found, _ = c
                return (found == 0) & (col < _B)

            def pfb_body(c, md=md, a8=a8, arvec=arvec):
                col, found, val = c
                pltpu.sync_copy(
                    pp16.at[pl.ds(a8, 8),
                            pl.ds(pl.multiple_of(col // 2, 128), 128)],
                    ptmp_v)
                runmin = jnp.full((16,), _NBIG, i32)
                for t in range(8):
                    vw = plsc.load_gather(ptmp_v, [arvec, t * 16 + iota])
                    runmin = jnp.minimum(runmin,
                                         match_keys(md, vw, col, t))
                fmin = jnp.min(runmin)
                f2 = (fmin < _NBIG).astype(i32)
                el = jnp.where(f2 == 1, (fmin - col) >> 1, 0)
                vw = plsc.load_gather(ptmp_v, [arvec,
                                               jnp.full((16,), el, i32)])
                v2 = jnp.where(fmin % 2 == 1, unpack_hi(vw), unpack_lo(vw))
                v2s = jnp.max(v2)
                return (col + 256, found | f2, jnp.where(f2 == 1, v2s, val))

            _, md["pfound"], md["pval"] = lax.while_loop(
                pfb_cond, pfb_body, (_PPREF, md["pfound"], md["pval"]))

            def nbody_once(cnt, vidx, want_diff=True, mylab=mylab,
                           gvec=md["gvec"], lvec=md["lvec"]):
                vlab = plsc.load_gather(lab_v, [vidx])
                m = (vlab != mylab) if want_diff else (vlab == mylab)
                rank = plsc.cumsum(m.astype(i32))
                sel = m & ((cnt + rank) <= _M)
                slot = jnp.where(sel, cnt + rank + 1, 2)
                plsc.store_scatter(cidx_v, [(gvec * 10 + slot) * 16 + lvec],
                                   vidx, mask=sel)
                return jnp.minimum(cnt + jnp.max(rank), _M)

            ncnt = nbody_once(jnp.int32(0),
                              plsc.load_gather(pnpre_v, [rvec * 16 + iota]))

            def nfb_cond(c):
                col, cnt = c
                return (cnt < _M) & (col < _B)

            def nfb_body(c, a8=a8, arvec=arvec, nbody_once=nbody_once):
                col, cnt = c
                pltpu.sync_copy(
                    pn.at[pl.ds(a8, 8),
                          pl.ds(pl.multiple_of(col, 128), 128)], ptmp_v)

                def inner(c2):
                    t, cnt2 = c2
                    vidx = plsc.load_gather(ptmp_v, [arvec, t * 16 + iota])
                    return (t + 1, nbody_once(cnt2, vidx))

                def inner_cond(c2):
                    t, cnt2 = c2
                    return (cnt2 < _M) & (t < 8)

                _, cnt = lax.while_loop(inner_cond, inner, (0, cnt))
                return (col + 128, cnt)

            ncnt0 = jnp.where(ncnt < _M, 0, ncnt)
            _, ncnt = lax.while_loop(nfb_cond, nfb_body, (0, ncnt0))
            anyneg = (ncnt > 0).astype(jnp.float32)

            def pad_cond(c):
                t, cnt = c
                return (cnt < _M) & (t < _B // 16)

            def pad_body(c, nbody_once=nbody_once):
                t, cnt = c
                return (t + 1, nbody_once(cnt, t * 16 + iota,
                                          want_diff=False))

            _, ncnt = lax.while_loop(pad_cond, pad_body, (0, ncnt))

            valid = md["pfound"].astype(jnp.float32) * anyneg
            lane0 = iota == 0
            plsc.store_scatter(cidx_v,
                               [(md["gvec"] * 10 + 1) * 16 + md["lvec"]],
                               jnp.full((16,), md["pval"], i32), mask=lane0)
            plsc.store_scatter(valid_v, [md["rvec"]],
                               jnp.full((16,), valid, jnp.float32),
                               mask=lane0)
        return carry

    lax.fori_loop(0, _R // 2, pair_fn, 0)

    zero16 = jnp.zeros((16,), jnp.float32)
    ngroups = _R // 16
    sems = (sem_a, sem_b)

    def fire(g):
        return [pltpu.async_copy(
            feats.at[cidx_v.at[pl.ds(g * 160 + h * 80, 80)]],
            gath_v.at[g % 2, pl.ds(h * 80, 80)], sems[g % 2])
            for h in range(2)]

    pending = {0: fire(0), 1: fire(1)}
    for g in range(ngroups):
        buf = g % 2
        for c in pending.pop(g):
            c.wait()

        rows = g * 16 + iota
        bufv = jnp.full((16,), buf, i32)

        def dbody(d, carry, _bufv=bufv):
            a2 = carry[0]
            accs = carry[1:10]
            c2s = carry[10:19]
            dv = jnp.full((16,), d, i32)
            a = plsc.load_gather(gath_v, [_bufv, iota, dv])
            out_accs = []
            out_c2s = []
            for k in range(9):
                b = plsc.load_gather(gath_v,
                                     [_bufv, (k + 1) * 16 + iota, dv])
                out_accs.append(accs[k] + a * b)
                out_c2s.append(c2s[k] + b * b)
            return (a2 + a * a, *out_accs, *out_c2s)

        res = lax.fori_loop(0, _D, dbody, tuple(zero16 for _ in range(19)))
        a2 = res[0]
        accs = res[1:10]
        c2s = res[10:19]

        ra = _rsqrt(jnp.maximum(a2, 1e-24))
        simv = [accs[k] * ra * _rsqrt(jnp.maximum(c2s[k], 1e-24))
                for k in range(9)]

        t1 = jnp.full((16,), -3.0e38, jnp.float32)
        t2 = t1
        t3 = t1
        for k in range(1, 9):
            v = simv[k]
            n1 = jnp.maximum(t1, v)
            v2 = jnp.minimum(t1, v)
            n2 = jnp.maximum(t2, v2)
            v3 = jnp.minimum(t2, v2)
            n3 = jnp.maximum(t3, v3)
            t1, t2, t3 = n1, n2, n3

        validv = plsc.load_gather(valid_v, [rows])
        cols = [simv[0], t1, t2, t3, validv]
        for c in range(_OW):
            vec = cols[c] if c < 5 else zero16
            plsc.store_scatter(outb_v, [iota, jnp.full((16,), c, i32)], vec)
        row0 = pl.multiple_of(base + g * 16, 16)
        pltpu.sync_copy(outb_v, out.at[pl.ds(row0, 16), :])

        if g + 2 < ngroups:
            pending[g + 2] = fire(g + 2)


_mesh = plsc.VectorSubcoreMesh(core_axis_name="c", subcore_axis_name="s",
                               num_cores=_NC, num_subcores=_NS)
_sc_select = pl.kernel(
    _sc_body,
    out_type=jax.ShapeDtypeStruct((_B, _OW), jnp.float32),
    mesh=_mesh,
    compiler_params=pltpu.CompilerParams(needs_layout_passes=False),
    scratch_types=[
        pltpu.VMEM((_B,), jnp.int32),
        pltpu.VMEM((_R, _PPREF // 2), jnp.int32),
        pltpu.VMEM((_R * 16,), jnp.int32),
        pltpu.VMEM((8, 128), jnp.int32),
        pltpu.VMEM(((_R // 16) * (_M + 2) * 16,), jnp.int32),
        pltpu.VMEM((_R,), jnp.float32),
        pltpu.VMEM((2, (_M + 2) * 16, _DP), jnp.float32),
        pltpu.VMEM((16, _OW), jnp.float32),
        pltpu.SemaphoreType.DMA,
        pltpu.SemaphoreType.DMA,
        pltpu.SemaphoreType.DMA,
    ],
)


def _loss_body(x_ref, o_ref):
    x = x_ref[...]
    l0 = x[:, 0:1] * _INV_T
    l1 = x[:, 1:2] * _INV_T
    l2 = x[:, 2:3] * _INV_T
    l3 = x[:, 3:4] * _INV_T
    v = x[:, 4:5]
    m = jnp.maximum(jnp.maximum(l0, l1), jnp.maximum(l2, l3))
    lse = m + jnp.log(jnp.exp(l0 - m) + jnp.exp(l1 - m)
                      + jnp.exp(l2 - m) + jnp.exp(l3 - m))
    losses = lse - l0
    nv = jnp.maximum(jnp.sum(v), 1.0)
    o_ref[...] = (jnp.sum(losses * v) / nv).reshape(1, 1)


_loss = pl.pallas_call(
    _loss_body,
    out_shape=jax.ShapeDtypeStruct((1, 1), jnp.float32),
)


def kernel(features, labels):
    labels = labels.reshape(-1).astype(jnp.int32)
    fpad = jnp.pad(features, ((0, 0), (0, _DP - _D)))
    sc = _sc_select(fpad, labels, _PP16, _PNC, _PN)
    return _loss(sc).reshape(())


# speedup vs baseline: 1.4827x; 1.0540x over previous
import jax
import jax.numpy as jnp
from jax import lax
from jax.experimental import pallas as pl
from jax.experimental.pallas import tpu as pltpu
from jax.experimental.pallas import tpu_sc as plsc

_B = 4096
_D = 64
_DP = 128
_M = 8
_INV_T = 2.0
_NC, _NS = 2, 16
_NW = _NC * _NS
_R = _B // _NW
_PPREF = 512
_OW = 16


def _threefry2x32(k0, k1, x0, x1):
    import numpy as np

    def rotl(x, r):
        return ((x << np.uint32(r)) | (x >> np.uint32(32 - r))).astype(np.uint32)

    ks0, ks1 = np.uint32(k0), np.uint32(k1)
    ks2 = np.uint32(ks0 ^ ks1 ^ np.uint32(0x1BD11BDA))
    rot1 = (13, 15, 26, 6)
    rot2 = (17, 29, 16, 24)
    x0 = (x0 + ks0).astype(np.uint32)
    x1 = (x1 + ks1).astype(np.uint32)

    def rounds(x0, x1, rots):
        for r in rots:
            x0 = (x0 + x1).astype(np.uint32)
            x1 = rotl(x1, r)
            x1 = (x1 ^ x0).astype(np.uint32)
        return x0, x1

    for i, (rots, ka, kb) in enumerate([
            (rot1, ks1, ks2), (rot2, ks2, ks0), (rot1, ks0, ks1),
            (rot2, ks1, ks2), (rot1, ks2, ks0)]):
        x0, x1 = rounds(x0, x1, rots)
        x0 = (x0 + ka).astype(np.uint32)
        x1 = (x1 + kb + np.uint32(i + 1)).astype(np.uint32)
    return x0, x1


def _np_gumbel(kd, n):
    import numpy as np

    i = np.arange(n, dtype=np.uint32)
    y0, y1 = _threefry2x32(kd[0], kd[1], np.zeros(n, np.uint32), i)
    bits = (y0 ^ y1).astype(np.uint32)
    fb = ((bits >> np.uint32(9)) | np.uint32(0x3F800000)).astype(np.uint32)
    f = fb.view(np.float32) - np.float32(1.0)
    tiny = np.float32(np.finfo(np.float32).tiny)
    u = np.maximum(tiny, f * (np.float32(1.0) - tiny) + tiny)
    return -np.log(-np.log(u))


def _perm_consts():
    import numpy as np

    kp = (1832780943, 270669613)
    kn = (64467757, 2916123636)
    gp = _np_gumbel(kp, _B * _B).reshape(_B, _B)
    gn = _np_gumbel(kn, _B * _B).reshape(_B, _B)
    pp = np.argsort(-gp, axis=1, kind="stable").astype(np.int32)
    pn = np.argsort(-gn, axis=1, kind="stable").astype(np.int32)
    pp16 = np.ascontiguousarray(pp).astype(np.uint16).view(np.int32)
    pnc = np.ascontiguousarray(pn[:, :16]).reshape(-1)
    return pp16, pnc, pn


_PP16, _PNC, _PN = _perm_consts()


def _rsqrt(x):
    i = plsc.bitcast(x, jnp.int32)
    y = plsc.bitcast(jnp.int32(0x5F3759DF) - (i >> 1), jnp.float32)
    for _ in range(3):
        y = y * (1.5 - 0.5 * x * y * y)
    return y


def _ln(x):
    bits = plsc.bitcast(x, jnp.int32)
    e = ((bits >> 23) - 127).astype(jnp.float32)
    mant = plsc.bitcast((bits & 0x7FFFFF) | 0x3F800000, jnp.float32)
    s = (mant - 1.0) / (mant + 1.0)
    s2 = s * s
    ln_m = 2.0 * s * (1.0 + s2 * (1.0 / 3.0 + s2 * (0.2 + s2 / 7.0)))
    return e * 0.6931471805599453 + ln_m


def _sc_body(feats, labels, pp16, pnc, pn, out,
             lab_v, ppre_v, pnpre_v, ptmp_v, cidx_v, valid_v,
             gath_v, outb_v, sem_a, sem_b, sem_c):
    i32 = jnp.int32
    iota = lax.iota(i32, 16)
    wid = lax.axis_index("s") * _NC + lax.axis_index("c")
    base = pl.multiple_of(wid * _R, _R)

    cps = [
        pltpu.async_copy(labels, lab_v, sem_c),
        pltpu.async_copy(pp16.at[pl.ds(base, _R), pl.ds(0, _PPREF // 2)],
                         ppre_v, sem_c),
        pltpu.async_copy(pnc.at[pl.ds(base * 16, _R * 16)], pnpre_v, sem_c),
    ]
    for c in cps:
        c.wait()

    for g in range(_R // 16):
        plsc.store_scatter(cidx_v, [g * 160 + iota], base + g * 16 + iota)

    _NBIG = jnp.int32(1 << 20)
    _SEG = 128

    def unpack_lo(v):
        return v & 0xFFFF

    def unpack_hi(v):
        return (v >> 16) & 0xFFFF

    def pair_fn(i, carry):
        rows_meta = []
        for s in range(2):
            r = 2 * i + s
            anchor = base + r
            meta = dict(
                r=r,
                anchor=anchor,
                avec=jnp.full((16,), anchor, i32),
                rvec=jnp.full((16,), r, i32),
                gvec=jnp.full((16,), r // 16, i32),
                lvec=jnp.full((16,), r % 16, i32),
            )
            meta["mylab"] = plsc.load_gather(lab_v, [meta["avec"]])
            rows_meta.append(meta)

        def match_keys(md, vwords, colbase, t):
            lo = unpack_lo(vwords)
            hi = unpack_hi(vwords)
            llo = plsc.load_gather(lab_v, [lo])
            lhi = plsc.load_gather(lab_v, [hi])
            mlo = (llo == md["mylab"]) & (lo != md["avec"])
            mhi = (lhi == md["mylab"]) & (hi != md["avec"])
            klo = jnp.where(mlo, colbase + 32 * t + 2 * iota, _NBIG)
            khi = jnp.where(mhi, colbase + 32 * t + 2 * iota + 1, _NBIG)
            return jnp.minimum(klo, khi)

        def seg_cond(c):
            seg, m0, m1 = c
            return (seg < _PPREF // _SEG) & ((m0 == _NBIG) | (m1 == _NBIG))

        def seg_body(c):
            seg, m0, m1 = c
            mins = [m0, m1]
            for s in range(2):
                md = rows_meta[s]
                runmin = jnp.full((16,), _NBIG, i32)
                for t in range(_SEG // 32):
                    vw = plsc.load_gather(
                        ppre_v, [md["rvec"], seg * (_SEG // 2) + t * 16 + iota])
                    runmin = jnp.minimum(runmin,
                                         match_keys(md, vw, seg * _SEG, t))
                mins[s] = jnp.minimum(mins[s], jnp.min(runmin))
            return (seg + 1, mins[0], mins[1])

        _, min0, min1 = lax.while_loop(seg_cond, seg_body, (0, _NBIG, _NBIG))

        for s, poscol in ((0, min0), (1, min1)):
            md = rows_meta[s]
            pfound = (poscol < _NBIG).astype(i32)
            e = jnp.where(pfound == 1, poscol >> 1, 0)
            vw = plsc.load_gather(ppre_v, [md["rvec"],
                                           jnp.full((16,), e, i32)])
            val = jnp.where(poscol % 2 == 1, unpack_hi(vw), unpack_lo(vw))
            md["pfound"] = pfound
            md["pval"] = jnp.where(pfound == 1, jnp.max(val), 0)

        for md in rows_meta:
            anchor = md["anchor"]
            mylab = md["mylab"]
            avec = md["avec"]
            rvec = md["rvec"]
            a8 = pl.multiple_of((anchor // 8) * 8, 8)
            arvec = jnp.full((16,), anchor % 8, i32)

            def pfb_cond(c):
                col, found, _ = c
                return (found == 0) & (col < _B)

            def pfb_body(c, md=md, a8=a8, arvec=arvec):
                col, found, val = c
                pltpu.sync_copy(
                    pp16.at[pl.ds(a8, 8),
                            pl.ds(pl.multiple_of(col // 2, 128), 128)],
                    ptmp_v)
                runmin = jnp.full((16,), _NBIG, i32)
                for t in range(8):
                    vw = plsc.load_gather(ptmp_v, [arvec, t * 16 + iota])
                    runmin = jnp.minimum(runmin,
                                         match_keys(md, vw, col, t))
                fmin = jnp.min(runmin)
                f2 = (fmin < _NBIG).astype(i32)
                el = jnp.where(f2 == 1, (fmin - col) >> 1, 0)
                vw = plsc.load_gather(ptmp_v, [arvec,
                                               jnp.full((16,), el, i32)])
                v2 = jnp.where(fmin % 2 == 1, unpack_hi(vw), unpack_lo(vw))
                v2s = jnp.max(v2)
                return (col + 256, found | f2, jnp.where(f2 == 1, v2s, val))

            _, md["pfound"], md["pval"] = lax.while_loop(
                pfb_cond, pfb_body, (_PPREF, md["pfound"], md["pval"]))

            def nbody_once(cnt, vidx, want_diff=True, mylab=mylab,
                           gvec=md["gvec"], lvec=md["lvec"]):
                vlab = plsc.load_gather(lab_v, [vidx])
                m = (vlab != mylab) if want_diff else (vlab == mylab)
                rank = plsc.cumsum(m.astype(i32))
                sel = m & ((cnt + rank) <= _M)
                slot = jnp.where(sel, cnt + rank + 1, 2)
                plsc.store_scatter(cidx_v, [(gvec * 10 + slot) * 16 + lvec],
                                   vidx, mask=sel)
                return jnp.minimum(cnt + jnp.max(rank), _M)

            ncnt = nbody_once(jnp.int32(0),
                              plsc.load_gather(pnpre_v, [rvec * 16 + iota]))

            def nfb_cond(c):
                col, cnt = c
                return (cnt < _M) & (col < _B)

            def nfb_body(c, a8=a8, arvec=arvec, nbody_once=nbody_once):
                col, cnt = c
                pltpu.sync_copy(
                    pn.at[pl.ds(a8, 8),
                          pl.ds(pl.multiple_of(col, 128), 128)], ptmp_v)

                def inner(c2):
                    t, cnt2 = c2
                    vidx = plsc.load_gather(ptmp_v, [arvec, t * 16 + iota])
                    return (t + 1, nbody_once(cnt2, vidx))

                def inner_cond(c2):
                    t, cnt2 = c2
                    return (cnt2 < _M) & (t < 8)

                _, cnt = lax.while_loop(inner_cond, inner, (0, cnt))
                return (col + 128, cnt)

            ncnt0 = jnp.where(ncnt < _M, 0, ncnt)
            _, ncnt = lax.while_loop(nfb_cond, nfb_body, (0, ncnt0))
            anyneg = (ncnt > 0).astype(jnp.float32)

            def pad_cond(c):
                t, cnt = c
                return (cnt < _M) & (t < _B // 16)

            def pad_body(c, nbody_once=nbody_once):
                t, cnt = c
                return (t + 1, nbody_once(cnt, t * 16 + iota,
                                          want_diff=False))

            _, ncnt = lax.while_loop(pad_cond, pad_body, (0, ncnt))

            valid = md["pfound"].astype(jnp.float32) * anyneg
            lane0 = iota == 0
            plsc.store_scatter(cidx_v,
                               [(md["gvec"] * 10 + 1) * 16 + md["lvec"]],
                               jnp.full((16,), md["pval"], i32), mask=lane0)
            plsc.store_scatter(valid_v, [md["rvec"]],
                               jnp.full((16,), valid, jnp.float32),
                               mask=lane0)
        return carry

    lax.fori_loop(0, _R // 2, pair_fn, 0)

    zero16 = jnp.zeros((16,), jnp.float32)
    ngroups = _R // 16
    sems = (sem_a, sem_b)

    def fire(g):
        return [pltpu.async_copy(
            feats.at[cidx_v.at[pl.ds(g * 160 + h * 80, 80)]],
            gath_v.at[g % 2, pl.ds(h * 80, 80)], sems[g % 2])
            for h in range(2)]

    pending = {0: fire(0), 1: fire(1)}
    lacc = zero16
    vacc = zero16
    for g in range(ngroups):
        buf = g % 2
        for c in pending.pop(g):
            c.wait()

        rows = g * 16 + iota
        bufv = jnp.full((16,), buf, i32)

        def dbody(d, carry, _bufv=bufv):
            a2 = carry[0]
            accs = carry[1:10]
            c2s = carry[10:19]
            dv = jnp.full((16,), d, i32)
            a = plsc.load_gather(gath_v, [_bufv, iota, dv])
            out_accs = []
            out_c2s = []
            for k in range(9):
                b = plsc.load_gather(gath_v,
                                     [_bufv, (k + 1) * 16 + iota, dv])
                out_accs.append(accs[k] + a * b)
                out_c2s.append(c2s[k] + b * b)
            return (a2 + a * a, *out_accs, *out_c2s)

        res = lax.fori_loop(0, _D, dbody, tuple(zero16 for _ in range(19)))
        a2 = res[0]
        accs = res[1:10]
        c2s = res[10:19]

        ra = _rsqrt(jnp.maximum(a2, 1e-24))
        simv = [accs[k] * ra * _rsqrt(jnp.maximum(c2s[k], 1e-24))
                for k in range(9)]

        t1 = jnp.full((16,), -3.0e38, jnp.float32)
        t2 = t1
        t3 = t1
        for k in range(1, 9):
            v = simv[k]
            n1 = jnp.maximum(t1, v)
            v2 = jnp.minimum(t1, v)
            n2 = jnp.maximum(t2, v2)
            v3 = jnp.minimum(t2, v2)
            n3 = jnp.maximum(t3, v3)
            t1, t2, t3 = n1, n2, n3

        validv = plsc.load_gather(valid_v, [rows])
        l0 = simv[0] * _INV_T
        l1 = t1 * _INV_T
        l2 = t2 * _INV_T
        l3 = t3 * _INV_T
        mx = jnp.maximum(jnp.maximum(l0, l1), jnp.maximum(l2, l3))
        ssum = (jnp.exp(l0 - mx) + jnp.exp(l1 - mx)
                + jnp.exp(l2 - mx) + jnp.exp(l3 - mx))
        losses = mx + _ln(ssum) - l0
        lacc = lacc + losses * validv
        vacc = vacc + validv

        if g + 2 < ngroups:
            pending[g + 2] = fire(g + 2)

    ls = jnp.sum(lacc)
    vc = jnp.sum(vacc)
    part = jnp.where(iota == 0, jnp.full((16,), ls, jnp.float32),
                     jnp.where(iota == 1, jnp.full((16,), vc, jnp.float32),
                               zero16))
    for rr in range(8):
        plsc.store_scatter(outb_v, [jnp.full((16,), rr, i32), iota],
                           part if rr == 0 else zero16)
    pltpu.sync_copy(outb_v, out.at[pl.ds(pl.multiple_of(wid * 8, 8), 8), :])


_mesh = plsc.VectorSubcoreMesh(core_axis_name="c", subcore_axis_name="s",
                               num_cores=_NC, num_subcores=_NS)
_sc_select = pl.kernel(
    _sc_body,
    out_type=jax.ShapeDtypeStruct((_NW * 8, _OW), jnp.float32),
    mesh=_mesh,
    compiler_params=pltpu.CompilerParams(needs_layout_passes=False),
    scratch_types=[
        pltpu.VMEM((_B,), jnp.int32),
        pltpu.VMEM((_R, _PPREF // 2), jnp.int32),
        pltpu.VMEM((_R * 16,), jnp.int32),
        pltpu.VMEM((8, 128), jnp.int32),
        pltpu.VMEM(((_R // 16) * (_M + 2) * 16,), jnp.int32),
        pltpu.VMEM((_R,), jnp.float32),
        pltpu.VMEM((2, (_M + 2) * 16, _DP), jnp.float32),
        pltpu.VMEM((8, _OW), jnp.float32),
        pltpu.SemaphoreType.DMA,
        pltpu.SemaphoreType.DMA,
        pltpu.SemaphoreType.DMA,
    ],
)


def _loss_body(x_ref, o_ref):
    x = x_ref[...]
    nv = jnp.maximum(jnp.sum(x[:, 1:2]), 1.0)
    o_ref[...] = (jnp.sum(x[:, 0:1]) / nv).reshape(1, 1)


_loss = pl.pallas_call(
    _loss_body,
    out_shape=jax.ShapeDtypeStruct((1, 1), jnp.float32),
)


def kernel(features, labels):
    labels = labels.reshape(-1).astype(jnp.int32)
    fpad = jnp.pad(features, ((0, 0), (0, _DP - _D)))
    parts = _sc_select(fpad, labels, _PP16, _PNC, _PN)
    return _loss(parts).reshape(())
